# baseline (speedup 1.0000x reference)
import functools

import jax
import jax.numpy as jnp
from jax import lax
from jax.experimental import pallas as pl
from jax.experimental.pallas import tpu as pltpu
from jax.experimental.pallas import tpu_sc as plsc

N = 10000
E = 640000
H = 2
C = 64
HC = H * C
L = 16
NT = 16
EPT = E // NT
CHUNK = 8000
NCHUNK = EPT // CHUNK
CAP_A = 256
SCAP = 272
CAP_B = 2048
UNR = 5
NEG = -1e30

W1L, W1R, WE1, ATT1, BS1, B1L, BIAS1, WE2, ATT2, B2L, B2R, BIAS2, BFC = range(13)

_IOTA = None


def _i32(v):
    return jnp.asarray(v, jnp.int32)


def _sload(ref, idx):
    return plsc.load_gather(ref, [jnp.full((L,), idx, jnp.int32)])[0]


def _sstore(ref, idx, val):
    iota = lax.iota(jnp.int32, L)
    plsc.store_scatter(ref, [jnp.full((L,), idx, jnp.int32)],
                       jnp.full((L,), val), mask=iota == 0)


def _sdiv(a, b):
    return (jnp.full((L,), a) / jnp.full((L,), b))[0]


def _append(ref, cc, x, mask, pos):
    plsc.store_scatter(ref, [cc + pos - 1], x, mask=mask)


def _body(src_h, dst_h, ea_h, x_h, tgt_h, wsm_h, w2l_h, w2r_h, wfc_h, out_h,
          xv, slotmap, dbuf, sbuf, ebuf, asrcb, aeab, asrcall, aeaall,
          cntsall, easall, s1, e2slot, e2ea, bslot, bsrc, bxs, bxd, bea,
          ba0, ba1, maxloc, dloc, sxloc, all16, amaxg, dg, sxg, w2lv,
          h1s, xl2acc, xr2acc, wbuf, rowbuf, tvec, pubi, pubf, wsm, wsc,
          sh_asrc, sh_aea, sh_cnt, sh_ea, sh_max, sh_d, sh_sx, sh_xl2):
    cid = lax.axis_index("c")
    t = lax.axis_index("s")
    on0 = cid == 0
    base = t * EPT
    zf = jnp.zeros((L,), jnp.float32)
    zi = jnp.zeros((L,), jnp.int32)

    @pl.when(on0)
    def _p1():
        pltpu.sync_copy(x_h, xv)
        pltpu.sync_copy(wsm_h, wsm)
        pltpu.sync_copy(w2l_h, w2lv)
        pltpu.sync_copy(tgt_h, pubi)
        tgtv = pubi[pl.ds(0, L)]
        for i in range(CAP_A // L):
            asrcb[pl.ds(i * L, L)] = zi
            aeab[pl.ds(i * L, L)] = zf

        def chunk_a(k, carry):
            cnt, easum = carry
            pltpu.sync_copy(dst_h.at[pl.ds(base + k * CHUNK, CHUNK)], dbuf)
            pltpu.sync_copy(src_h.at[pl.ds(base + k * CHUNK, CHUNK)], sbuf)
            pltpu.sync_copy(ea_h.at[pl.ds(base + k * CHUNK, CHUNK)], ebuf)

            @plsc.parallel_loop(0, CHUNK // L, 1, unroll=8, carry=(cnt, easum))
            def it(i, c2):
                cnt2, es = c2
                d = dbuf[pl.ds(i * L, L)]
                sv = sbuf[pl.ds(i * L, L)]
                ev = ebuf[pl.ds(i * L, L)]
                m = d == tgtv
                cc = jnp.minimum(cnt2, CAP_A - L)
                pos = plsc.cumsum(m.astype(jnp.int32))
                _append(asrcb, cc, sv, m, pos)
                _append(aeab, cc, ev, m, pos)
                return cnt2 + pos[L - 1], es + ev

            return it

        cnt, easum = lax.fori_loop(0, NCHUNK, chunk_a, (_i32(0), zf))
        cnt = jnp.minimum(cnt, CAP_A)
        pltpu.sync_copy(asrcb, sh_asrc.at[pl.ds(t * CAP_A, CAP_A)])
        pltpu.sync_copy(aeab, sh_aea.at[pl.ds(t * CAP_A, CAP_A)])
        pubi[pl.ds(0, L)] = jnp.full((L,), cnt, jnp.int32)
        pltpu.sync_copy(pubi, sh_cnt.at[pl.ds(t * L, L)])
        pubf[pl.ds(0, L)] = easum
        pltpu.sync_copy(pubf, sh_ea.at[pl.ds(t * L, L)])

    plsc.subcore_barrier()

    @pl.when(on0)
    def _p2():
        pltpu.sync_copy(sh_asrc, asrcall)
        pltpu.sync_copy(sh_aea, aeaall)
        pltpu.sync_copy(sh_cnt, cntsall)
        pltpu.sync_copy(sh_ea, easall)
        pltpu.sync_copy(tgt_h, pubi)
        tgt = pubi[pl.ds(0, L)][0]

        acc = zf
        for t2 in range(NT):
            acc = acc + easall[pl.ds(t2 * L, L)]
        ea_mean = jnp.sum(acc) * (1.0 / E)

        for i in range(5 * HC // L):
            wv = wsm[pl.ds(i * L, L)]
            for l in range(L):
                wsc[i * L + l] = wv[l]

        def zmap(i, _):
            slotmap[pl.ds(i * L, L)] = zi
            return 0

        lax.fori_loop(0, N // L, zmap, 0)
        for i in range(SCAP // L):
            s1[pl.ds(i * L, L)] = zi

        _sstore(slotmap, tgt, _i32(1))
        _sstore(s1, 0, tgt)
        _sstore(e2slot, 0, _i32(0))
        _sstore(e2ea, 0, ea_mean)
        nslots = _i32(1)
        ne2 = _i32(1)
        for t2 in range(NT):
            cnt_t = _sload(cntsall, t2 * L)

            def ded(j, carry):
                ns, ne = carry
                srcv = _sload(asrcall, t2 * CAP_A + j)
                eav = _sload(aeaall, t2 * CAP_A + j)
                sl = _sload(slotmap, srcv)

                @pl.when(sl == 0)
                def _new():
                    _sstore(slotmap, srcv, jnp.minimum(ns + 1, SCAP))
                    _sstore(s1, jnp.minimum(ns, SCAP - 1), srcv)

                slot_j = jnp.where(sl == 0, ns, sl - 1)
                nec = jnp.minimum(ne, SCAP - 1)
                _sstore(e2slot, nec, jnp.minimum(slot_j, SCAP - 1))
                _sstore(e2ea, nec, eav)
                return jnp.where(sl == 0, ns + 1, ns), ne + 1

            nslots, ne2 = lax.fori_loop(0, cnt_t, ded, (nslots, ne2))
        nslots = jnp.minimum(nslots, SCAP)
        ne2 = jnp.minimum(ne2, SCAP)
        _sstore(pubi, 0, nslots)
        _sstore(pubi, 1, ne2)

        for i in range(CAP_B // L):
            bslot[pl.ds(i * L, L)] = jnp.full((L,), 1, jnp.int32)
            bsrc[pl.ds(i * L, L)] = zi

        def chunk_b(k, cnt):
            pltpu.sync_copy(dst_h.at[pl.ds(base + k * CHUNK, CHUNK)], dbuf)
            pltpu.sync_copy(src_h.at[pl.ds(base + k * CHUNK, CHUNK)], sbuf)
            pltpu.sync_copy(ea_h.at[pl.ds(base + k * CHUNK, CHUNK)], ebuf)

            @plsc.parallel_loop(0, CHUNK // L, 1, unroll=8, carry=cnt)
            def it(i, cnt2):
                d = dbuf[pl.ds(i * L, L)]
                slv = plsc.load_gather(slotmap, [d])
                m = slv > 0
                sv = sbuf[pl.ds(i * L, L)]
                ev = ebuf[pl.ds(i * L, L)]
                cc = jnp.minimum(cnt2, CAP_B - L)
                pos = plsc.cumsum(m.astype(jnp.int32))
                _append(bslot, cc, slv, m, pos)
                _append(bsrc, cc, sv, m, pos)
                _append(bea, cc, ev, m, pos)
                return cnt2 + pos[L - 1]

            return it

        cntb = lax.fori_loop(0, NCHUNK, chunk_b, _i32(0))
        cntb = jnp.minimum(cntb, CAP_B)

        n_my = (nslots + NT - 1 - t) // NT

        def selfrec(k, _):
            s = t + k * NT
            j = jnp.minimum(cntb + k, CAP_B - 1)
            _sstore(bslot, j, s + 1)
            _sstore(bsrc, j, _sload(s1, s))
            _sstore(bea, j, ea_mean)
            return 0

        lax.fori_loop(0, n_my, selfrec, 0)
        cntb = jnp.minimum(cntb + n_my, CAP_B)
        _sstore(pubi, 2, cntb)
        nb = (cntb + L - 1) // L

        def gxy(r, _):
            sv = bsrc[pl.ds(r * L, L)]
            slv = jnp.minimum(bslot[pl.ds(r * L, L)] - 1, SCAP - 1)
            dn = plsc.load_gather(s1, [slv])
            bxs[pl.ds(r * L, L)] = plsc.load_gather(xv, [sv])
            bxd[pl.ds(r * L, L)] = plsc.load_gather(xv, [dn])
            return 0

        lax.fori_loop(0, nb, gxy, 0)

        def alpha1(r, _):
            xs = bxs[pl.ds(r * L, L)]
            xd = bxd[pl.ds(r * L, L)]
            ev = bea[pl.ds(r * L, L)]

            def inner(c, accs):
                a0, a1 = accs
                m0 = xs * wsc[W1L * HC + c] + xd * wsc[W1R * HC + c] \
                    + ev * wsc[WE1 * HC + c] + wsc[BS1 * HC + c]
                m0 = jnp.where(m0 >= 0, m0, 0.2 * m0)
                m1 = xs * wsc[W1L * HC + C + c] + xd * wsc[W1R * HC + C + c] \
                    + ev * wsc[WE1 * HC + C + c] + wsc[BS1 * HC + C + c]
                m1 = jnp.where(m1 >= 0, m1, 0.2 * m1)
                return (a0 + m0 * wsc[ATT1 * HC + c],
                        a1 + m1 * wsc[ATT1 * HC + C + c])

            a0, a1 = lax.fori_loop(0, C, inner, (zf, zf))
            ba0[pl.ds(r * L, L)] = a0
            ba1[pl.ds(r * L, L)] = a1
            return 0

        lax.fori_loop(0, nb, alpha1, 0)

        for i in range(SCAP * 2 // L):
            maxloc[pl.ds(i * L, L)] = jnp.full((L,), NEG, jnp.float32)

        def mx(j, _):
            i0 = jnp.minimum(_sload(bslot, j) - 1, SCAP - 1) * 2
            _sstore(maxloc, i0, jnp.maximum(_sload(maxloc, i0), _sload(ba0, j)))
            _sstore(maxloc, i0 + 1,
                    jnp.maximum(_sload(maxloc, i0 + 1), _sload(ba1, j)))
            return 0

        lax.fori_loop(0, cntb, mx, 0)
        pltpu.sync_copy(maxloc, sh_max.at[pl.ds(t * SCAP * 2, SCAP * 2)])

    plsc.subcore_barrier()

    @pl.when(on0)
    def _p4():
        hdr = pubi[pl.ds(0, L)]
        nslots = hdr[0]
        cntb = hdr[2]
        nb = (cntb + L - 1) // L
        nbm = (nslots * 2 + L - 1) // L
        pltpu.sync_copy(sh_max, all16)

        def cmax(i, _):
            acc = jnp.full((L,), NEG, jnp.float32)
            for t2 in range(NT):
                acc = jnp.maximum(acc, all16[pl.ds(t2 * SCAP * 2 + i * L, L)])
            amaxg[pl.ds(i * L, L)] = acc
            return 0

        lax.fori_loop(0, nbm, cmax, 0)

        def expp(r, _):
            slv = jnp.minimum(bslot[pl.ds(r * L, L)] - 1, SCAP - 1)
            i0 = slv * 2
            am0 = plsc.load_gather(amaxg, [i0])
            am1 = plsc.load_gather(amaxg, [i0 + 1])
            ba0[pl.ds(r * L, L)] = jnp.exp(ba0[pl.ds(r * L, L)] - am0)
            ba1[pl.ds(r * L, L)] = jnp.exp(ba1[pl.ds(r * L, L)] - am1)
            return 0

        lax.fori_loop(0, nb, expp, 0)
        for i in range(SCAP * 2 // L):
            dloc[pl.ds(i * L, L)] = zf
            sxloc[pl.ds(i * L, L)] = zf

        def accj(j, _):
            i0 = jnp.minimum(_sload(bslot, j) - 1, SCAP - 1) * 2
            e0 = _sload(ba0, j)
            e1 = _sload(ba1, j)
            xsj = _sload(bxs, j)
            _sstore(dloc, i0, _sload(dloc, i0) + e0)
            _sstore(dloc, i0 + 1, _sload(dloc, i0 + 1) + e1)
            _sstore(sxloc, i0, _sload(sxloc, i0) + e0 * xsj)
            _sstore(sxloc, i0 + 1, _sload(sxloc, i0 + 1) + e1 * xsj)
            return 0

        lax.fori_loop(0, cntb, accj, 0)
        pltpu.sync_copy(dloc, sh_d.at[pl.ds(t * SCAP * 2, SCAP * 2)])
        pltpu.sync_copy(sxloc, sh_sx.at[pl.ds(t * SCAP * 2, SCAP * 2)])

    plsc.subcore_barrier()

    @pl.when(on0)
    def _p5():
        nslots = pubi[pl.ds(0, L)][0]
        nbm = (nslots * 2 + L - 1) // L
        pltpu.sync_copy(sh_d, all16)

        def csum_d(i, _):
            acc = zf
            for t2 in range(NT):
                acc = acc + all16[pl.ds(t2 * SCAP * 2 + i * L, L)]
            dg[pl.ds(i * L, L)] = acc
            return 0

        lax.fori_loop(0, nbm, csum_d, 0)
        pltpu.sync_copy(sh_sx, all16)

        def csum_s(i, _):
            acc = zf
            for t2 in range(NT):
                acc = acc + all16[pl.ds(t2 * SCAP * 2 + i * L, L)]
            sxg[pl.ds(i * L, L)] = acc
            return 0

        lax.fori_loop(0, nbm, csum_s, 0)

        n_my = (nslots + NT - 1 - t) // NT

        def slotk(k, _):
            s = t + k * NT
            i0 = s * 2
            d0 = _sload(dg, i0)
            d1 = _sload(dg, i0 + 1)
            sx0 = _sdiv(_sload(sxg, i0), d0 + 1e-16)
            sx1 = _sdiv(_sload(sxg, i0 + 1), d1 + 1e-16)
            sa0 = _sdiv(d0, d0 + 1e-16)
            sa1 = _sdiv(d1, d1 + 1e-16)
            for k2 in range(8):
                sx = sx0 if k2 < 4 else sx1
                sa = sa0 if k2 < 4 else sa1
                hv = sx * wsm[pl.ds(W1L * HC + k2 * L, L)] \
                    + sa * wsm[pl.ds(B1L * HC + k2 * L, L)] \
                    + wsm[pl.ds(BIAS1 * HC + k2 * L, L)]
                h1s[pl.ds(k * HC + k2 * L, L)] = jnp.maximum(hv, 0.0)

            def mv(c, accs):
                hval = _sload(h1s, k * HC + c)
                return tuple(accs[k2] + hval * w2lv[pl.ds(c * HC + k2 * L, L)]
                             for k2 in range(8))

            accs = tuple(wsm[pl.ds(B2L * HC + k2 * L, L)] for k2 in range(8))
            accs = lax.fori_loop(0, HC, mv, accs)
            for k2 in range(8):
                xl2acc[pl.ds(k * HC + k2 * L, L)] = accs[k2]
            pltpu.sync_copy(xl2acc.at[pl.ds(k * HC, HC)],
                            sh_xl2.at[pl.ds(s * HC, HC)])
            return 0

        lax.fori_loop(0, n_my, slotk, 0)

        @pl.when(t == 0)
        def _xr():
            accs = tuple(wsm[pl.ds(B2R * HC + k2 * L, L)] for k2 in range(8))
            for q in range(8):
                pltpu.sync_copy(w2r_h.at[pl.ds(q * L * HC, L * HC)], wbuf)

                def mv2(rr, a2):
                    hval = _sload(h1s, q * L + rr)
                    return tuple(a2[k2] + hval * wbuf[pl.ds(rr * HC + k2 * L, L)]
                                 for k2 in range(8))

                accs = lax.fori_loop(0, L, mv2, accs)
            for k2 in range(8):
                xr2acc[pl.ds(k2 * L, L)] = accs[k2]

    plsc.subcore_barrier()

    @pl.when(jnp.logical_and(on0, t == 0))
    def _p6():
        ne2 = pubi[pl.ds(0, L)][1]

        def a2j(j, carry):
            m20, m21 = carry
            s = jnp.minimum(_sload(e2slot, j), SCAP - 1)
            eav = _sload(e2ea, j)
            pltpu.sync_copy(sh_xl2.at[pl.ds(s * HC, HC)], rowbuf)
            acc0 = zf
            acc1 = zf
            for k2 in range(8):
                mv = rowbuf[pl.ds(k2 * L, L)] + xr2acc[pl.ds(k2 * L, L)] \
                    + eav * wsm[pl.ds(WE2 * HC + k2 * L, L)]
                mv = jnp.where(mv >= 0, mv, 0.2 * mv)
                prod = mv * wsm[pl.ds(ATT2 * HC + k2 * L, L)]
                if k2 < 4:
                    acc0 = acc0 + prod
                else:
                    acc1 = acc1 + prod
            a0 = jnp.sum(acc0)
            a1 = jnp.sum(acc1)
            _sstore(ba0, j, a0)
            _sstore(ba1, j, a1)
            return jnp.maximum(m20, a0), jnp.maximum(m21, a1)

        m20, m21 = lax.fori_loop(0, ne2, a2j,
                                 (jnp.float32(NEG), jnp.float32(NEG)))

        def e2j(j, carry):
            d20, d21 = carry
            e0 = jnp.exp(jnp.full((L,), _sload(ba0, j) - m20))[0]
            e1 = jnp.exp(jnp.full((L,), _sload(ba1, j) - m21))[0]
            _sstore(ba0, j, e0)
            _sstore(ba1, j, e1)
            return d20 + e0, d21 + e1

        d20, d21 = lax.fori_loop(0, ne2, e2j,
                                 (jnp.float32(0), jnp.float32(0)))
        inv0 = _sdiv(jnp.float32(1.0), d20 + 1e-16)
        inv1 = _sdiv(jnp.float32(1.0), d21 + 1e-16)

        def wsumj(j, accs):
            s = jnp.minimum(_sload(e2slot, j), SCAP - 1)
            pltpu.sync_copy(sh_xl2.at[pl.ds(s * HC, HC)], rowbuf)
            w0 = _sload(ba0, j) * inv0
            w1 = _sload(ba1, j) * inv1
            return tuple(accs[k2] + (w0 if k2 < 4 else w1)
                         * rowbuf[pl.ds(k2 * L, L)] for k2 in range(8))

        accs = tuple(wsm[pl.ds(BIAS2 * HC + k2 * L, L)] for k2 in range(8))
        accs = lax.fori_loop(0, ne2, wsumj, accs)
        for k2 in range(8):
            tvec[pl.ds(k2 * L, L)] = accs[k2]

        raccs = tuple(wsm[pl.ds(BFC * HC + k2 * L, L)] for k2 in range(8))
        for q in range(8):
            pltpu.sync_copy(wfc_h.at[pl.ds(q * L * HC, L * HC)], wbuf)

            def mv3(rr, a2):
                hval = _sload(tvec, q * L + rr)
                return tuple(a2[k2] + hval * wbuf[pl.ds(rr * HC + k2 * L, L)]
                             for k2 in range(8))

            raccs = lax.fori_loop(0, L, mv3, raccs)
        for k2 in range(8):
            rowbuf[pl.ds(k2 * L, L)] = raccs[k2]
        pltpu.sync_copy(rowbuf, out_h)


@jax.jit
def _run(src, dst, ea, x, tgtv, wsmall, w2l, w2r, wfc):
    f32 = jnp.float32
    i32 = jnp.int32
    kfn = pl.kernel(
        _body,
        out_type=jax.ShapeDtypeStruct((HC,), f32),
        compiler_params=pltpu.CompilerParams(
            needs_layout_passes=False, use_tc_tiling_on_sc=False),
        mesh=plsc.VectorSubcoreMesh(core_axis_name="c", subcore_axis_name="s"),
        scratch_types=[
            pltpu.VMEM((N,), f32),
            pltpu.VMEM((N,), i32),
            pltpu.VMEM((CHUNK,), i32),
            pltpu.VMEM((CHUNK,), i32),
            pltpu.VMEM((CHUNK,), f32),
            pltpu.VMEM((CAP_A,), i32),
            pltpu.VMEM((CAP_A,), f32),
            pltpu.VMEM((NT * CAP_A,), i32),
            pltpu.VMEM((NT * CAP_A,), f32),
            pltpu.VMEM((NT * L,), i32),
            pltpu.VMEM((NT * L,), f32),
            pltpu.VMEM((SCAP,), i32),
            pltpu.VMEM((SCAP,), i32),
            pltpu.VMEM((SCAP,), f32),
            pltpu.VMEM((CAP_B,), i32),
            pltpu.VMEM((CAP_B,), i32),
            pltpu.VMEM((CAP_B,), f32),
            pltpu.VMEM((CAP_B,), f32),
            pltpu.VMEM((CAP_B,), f32),
            pltpu.VMEM((CAP_B,), f32),
            pltpu.VMEM((CAP_B,), f32),
            pltpu.VMEM((SCAP * 2,), f32),
            pltpu.VMEM((SCAP * 2,), f32),
            pltpu.VMEM((SCAP * 2,), f32),
            pltpu.VMEM((NT * SCAP * 2,), f32),
            pltpu.VMEM((SCAP * 2,), f32),
            pltpu.VMEM((SCAP * 2,), f32),
            pltpu.VMEM((SCAP * 2,), f32),
            pltpu.VMEM((HC * HC,), f32),
            pltpu.VMEM((17 * HC,), f32),
            pltpu.VMEM((17 * HC,), f32),
            pltpu.VMEM((HC,), f32),
            pltpu.VMEM((L * HC,), f32),
            pltpu.VMEM((HC,), f32),
            pltpu.VMEM((HC,), f32),
            pltpu.VMEM((L,), i32),
            pltpu.VMEM((L,), f32),
            pltpu.VMEM((13 * HC,), f32),
            pltpu.SMEM((5 * HC,), f32),
            pltpu.VMEM_SHARED((NT * CAP_A,), i32),
            pltpu.VMEM_SHARED((NT * CAP_A,), f32),
            pltpu.VMEM_SHARED((NT * L,), i32),
            pltpu.VMEM_SHARED((NT * L,), f32),
            pltpu.VMEM_SHARED((NT * SCAP * 2,), f32),
            pltpu.VMEM_SHARED((NT * SCAP * 2,), f32),
            pltpu.VMEM_SHARED((NT * SCAP * 2,), f32),
            pltpu.VMEM_SHARED((SCAP * HC,), f32),
        ],
    )
    return kfn(src, dst, ea, x, tgtv, wsmall, w2l, w2r, wfc)


def kernel(x, edge_index, edge_attr, target_node_idx, W1l, b1l, W1r, b1r,
           We1, att1, bias1, W2l, b2l, W2r, b2r, We2, att2, bias2, Wfc, bfc):
    src = edge_index[0]
    dst = edge_index[1]
    xf = x.reshape(N).astype(jnp.float32)
    eaf = edge_attr.reshape(E).astype(jnp.float32)
    tgtv = jnp.full((L,), jnp.asarray(target_node_idx, jnp.int32))
    wsmall = jnp.concatenate([
        W1l.reshape(HC), W1r.reshape(HC), We1.reshape(HC), att1.reshape(HC),
        (b1l + b1r).reshape(HC), b1l.reshape(HC), bias1.reshape(HC),
        We2.reshape(HC), att2.reshape(HC), b2l.reshape(HC), b2r.reshape(HC),
        bias2.reshape(HC), bfc.reshape(HC)]).astype(jnp.float32)
    return _run(src.astype(jnp.int32), dst.astype(jnp.int32), eaf, xf, tgtv,
                wsmall, W2l.reshape(HC * HC).astype(jnp.float32),
                W2r.reshape(HC * HC).astype(jnp.float32),
                Wfc.reshape(HC * HC).astype(jnp.float32))


# speedup vs baseline: 1.0051x; 1.0051x over previous
import functools

import jax
import jax.numpy as jnp
from jax import lax
from jax.experimental import pallas as pl
from jax.experimental.pallas import tpu as pltpu
from jax.experimental.pallas import tpu_sc as plsc

N = 10000
E = 640000
H = 2
C = 64
HC = H * C
L = 16
NT = 16
EPT = E // NT
CHUNK = 8000
NCHUNK = EPT // CHUNK
CAP_A = 256
SCAP = 272
CAP_B = 2048
NEG = -1e30

W1L, W1R, WE1, ATT1, BS1, B1L, BIAS1, WE2, ATT2, B2L, B2R, BIAS2, BFC = range(13)


def _i32(v):
    return jnp.asarray(v, jnp.int32)


def _sload(ref, idx):
    return plsc.load_gather(ref, [jnp.full((L,), idx, jnp.int32)])[0]


def _sstore(ref, idx, val):
    iota = lax.iota(jnp.int32, L)
    plsc.store_scatter(ref, [jnp.full((L,), idx, jnp.int32)],
                       jnp.full((L,), val), mask=iota == 0)


def _sdiv(a, b):
    return (jnp.full((L,), a) / jnp.full((L,), b))[0]


def _append(ref, cc, x, mask, pos):
    plsc.store_scatter(ref, [cc + pos - 1], x, mask=mask)


def _body(src_h, dst_h, ea_h, x_h, tgt_h, wsm_h, w2l_h, w2r_h, wfc_h, out_h,
          xv, slotmap, dbuf, sbuf, ebuf, asrcb, aeab, asrcall, aeaall,
          cntsall, easall, s1, e2slot, e2ea, bslot, bsrc, bxs, bxd, bea,
          ba0, ba1, maxloc, dloc, sxloc, all16, amaxg, dg, sxg, w2lv,
          h1s, xl2acc, xr2acc, wbuf, rowbuf, tvec, pubi, pubf, wsm, wsc,
          sh_asrc, sh_aea, sh_cnt, sh_ea, sh_max, sh_d, sh_sx, sh_xl2):
    cid = lax.axis_index("c")
    t = lax.axis_index("s")
    on0 = cid == 0
    base = t * EPT
    zf = jnp.zeros((L,), jnp.float32)
    zi = jnp.zeros((L,), jnp.int32)

    @pl.when(on0)
    def _p1():
        pltpu.sync_copy(x_h, xv)
        pltpu.sync_copy(wsm_h, wsm)
        pltpu.sync_copy(w2l_h, w2lv)
        pltpu.sync_copy(tgt_h, pubi)
        tgtv = pubi[pl.ds(0, L)]
        for i in range(CAP_A // L):
            asrcb[pl.ds(i * L, L)] = zi
            aeab[pl.ds(i * L, L)] = zf

        def chunk_a(k, carry):
            cnt, easum = carry
            pltpu.sync_copy(dst_h.at[pl.ds(base + k * CHUNK, CHUNK)], dbuf)
            pltpu.sync_copy(src_h.at[pl.ds(base + k * CHUNK, CHUNK)], sbuf)
            pltpu.sync_copy(ea_h.at[pl.ds(base + k * CHUNK, CHUNK)], ebuf)

            @plsc.parallel_loop(0, CHUNK // L, 1, unroll=4, carry=(cnt, easum))
            def it(i, c2):
                cnt2, es = c2
                d = dbuf[pl.ds(i * L, L)]
                sv = sbuf[pl.ds(i * L, L)]
                ev = ebuf[pl.ds(i * L, L)]
                m = d == tgtv
                cc = jnp.minimum(cnt2, CAP_A - L)
                pos = plsc.cumsum(m.astype(jnp.int32))
                _append(asrcb, cc, sv, m, pos)
                _append(aeab, cc, ev, m, pos)
                return cnt2 + pos[L - 1], es + ev

            return it

        cnt, easum = lax.fori_loop(0, NCHUNK, chunk_a, (_i32(0), zf))
        cnt = jnp.minimum(cnt, CAP_A)
        pltpu.sync_copy(asrcb, sh_asrc.at[pl.ds(t * CAP_A, CAP_A)])
        pltpu.sync_copy(aeab, sh_aea.at[pl.ds(t * CAP_A, CAP_A)])
        pubi[pl.ds(0, L)] = jnp.full((L,), cnt, jnp.int32)
        pltpu.sync_copy(pubi, sh_cnt.at[pl.ds(t * L, L)])
        pubf[pl.ds(0, L)] = easum
        pltpu.sync_copy(pubf, sh_ea.at[pl.ds(t * L, L)])

    plsc.subcore_barrier()

    @pl.when(on0)
    def _p2():
        pltpu.sync_copy(sh_asrc, asrcall)
        pltpu.sync_copy(sh_aea, aeaall)
        pltpu.sync_copy(sh_cnt, cntsall)
        pltpu.sync_copy(sh_ea, easall)
        pltpu.sync_copy(tgt_h, pubi)
        tgt = pubi[pl.ds(0, L)][0]

        acc = zf
        for t2 in range(NT):
            acc = acc + easall[pl.ds(t2 * L, L)]
        ea_mean = jnp.sum(acc) * (1.0 / E)

        for i in range(5 * HC // L):
            wv = wsm[pl.ds(i * L, L)]
            for l in range(L):
                wsc[i * L + l] = wv[l]

        def zmap(i, _):
            slotmap[pl.ds(i * L, L)] = zi
            return 0

        lax.fori_loop(0, N // L, zmap, 0)
        for i in range(SCAP // L):
            s1[pl.ds(i * L, L)] = zi

        _sstore(slotmap, tgt, _i32(1))
        _sstore(s1, 0, tgt)
        _sstore(e2slot, 0, _i32(0))
        _sstore(e2ea, 0, ea_mean)
        nslots = _i32(1)
        ne2 = _i32(1)
        for t2 in range(NT):
            cnt_t = _sload(cntsall, t2 * L)

            def ded(j, carry):
                ns, ne = carry
                srcv = _sload(asrcall, t2 * CAP_A + j)
                eav = _sload(aeaall, t2 * CAP_A + j)
                sl = _sload(slotmap, srcv)

                @pl.when(sl == 0)
                def _new():
                    _sstore(slotmap, srcv, jnp.minimum(ns + 1, SCAP))
                    _sstore(s1, jnp.minimum(ns, SCAP - 1), srcv)

                slot_j = jnp.where(sl == 0, ns, sl - 1)
                nec = jnp.minimum(ne, SCAP - 1)
                _sstore(e2slot, nec, jnp.minimum(slot_j, SCAP - 1))
                _sstore(e2ea, nec, eav)
                return jnp.where(sl == 0, ns + 1, ns), ne + 1

            nslots, ne2 = lax.fori_loop(0, cnt_t, ded, (nslots, ne2))
        nslots = jnp.minimum(nslots, SCAP)
        ne2 = jnp.minimum(ne2, SCAP)
        _sstore(pubi, 0, nslots)
        _sstore(pubi, 1, ne2)

        for i in range(CAP_B // L):
            bslot[pl.ds(i * L, L)] = jnp.full((L,), 1, jnp.int32)
            bsrc[pl.ds(i * L, L)] = zi

        def chunk_b(k, cnt):
            pltpu.sync_copy(dst_h.at[pl.ds(base + k * CHUNK, CHUNK)], dbuf)
            pltpu.sync_copy(src_h.at[pl.ds(base + k * CHUNK, CHUNK)], sbuf)
            pltpu.sync_copy(ea_h.at[pl.ds(base + k * CHUNK, CHUNK)], ebuf)

            @plsc.parallel_loop(0, CHUNK // L, 1, unroll=4, carry=cnt)
            def it(i, cnt2):
                d = dbuf[pl.ds(i * L, L)]
                slv = plsc.load_gather(slotmap, [d])
                m = slv > 0
                sv = sbuf[pl.ds(i * L, L)]
                ev = ebuf[pl.ds(i * L, L)]
                cc = jnp.minimum(cnt2, CAP_B - L)
                pos = plsc.cumsum(m.astype(jnp.int32))
                _append(bslot, cc, slv, m, pos)
                _append(bsrc, cc, sv, m, pos)
                _append(bea, cc, ev, m, pos)
                return cnt2 + pos[L - 1]

            return it

        cntb = lax.fori_loop(0, NCHUNK, chunk_b, _i32(0))
        cntb = jnp.minimum(cntb, CAP_B)

        n_my = (nslots + NT - 1 - t) // NT

        def selfrec(k, _):
            s = t + k * NT
            j = jnp.minimum(cntb + k, CAP_B - 1)
            _sstore(bslot, j, s + 1)
            _sstore(bsrc, j, _sload(s1, s))
            _sstore(bea, j, ea_mean)
            return 0

        lax.fori_loop(0, n_my, selfrec, 0)
        cntb = jnp.minimum(cntb + n_my, CAP_B)
        _sstore(pubi, 2, cntb)
        nb = (cntb + L - 1) // L

        def gxy(r, _):
            sv = bsrc[pl.ds(r * L, L)]
            slv = jnp.minimum(bslot[pl.ds(r * L, L)] - 1, SCAP - 1)
            dn = plsc.load_gather(s1, [slv])
            bxs[pl.ds(r * L, L)] = plsc.load_gather(xv, [sv])
            bxd[pl.ds(r * L, L)] = plsc.load_gather(xv, [dn])
            return 0

        lax.fori_loop(0, nb, gxy, 0)

        def alpha1(r, _):
            xs = bxs[pl.ds(r * L, L)]
            xd = bxd[pl.ds(r * L, L)]
            ev = bea[pl.ds(r * L, L)]

            def inner(c, accs):
                a0, a1 = accs
                m0 = xs * wsc[W1L * HC + c] + xd * wsc[W1R * HC + c] \
                    + ev * wsc[WE1 * HC + c] + wsc[BS1 * HC + c]
                m0 = jnp.where(m0 >= 0, m0, 0.2 * m0)
                m1 = xs * wsc[W1L * HC + C + c] + xd * wsc[W1R * HC + C + c] \
                    + ev * wsc[WE1 * HC + C + c] + wsc[BS1 * HC + C + c]
                m1 = jnp.where(m1 >= 0, m1, 0.2 * m1)
                return (a0 + m0 * wsc[ATT1 * HC + c],
                        a1 + m1 * wsc[ATT1 * HC + C + c])

            a0, a1 = lax.fori_loop(0, C, inner, (zf, zf))
            ba0[pl.ds(r * L, L)] = a0
            ba1[pl.ds(r * L, L)] = a1
            return 0

        lax.fori_loop(0, nb, alpha1, 0)

        for i in range(SCAP * 2 // L):
            maxloc[pl.ds(i * L, L)] = jnp.full((L,), NEG, jnp.float32)

        def mx(j, _):
            i0 = jnp.minimum(_sload(bslot, j) - 1, SCAP - 1) * 2
            _sstore(maxloc, i0, jnp.maximum(_sload(maxloc, i0), _sload(ba0, j)))
            _sstore(maxloc, i0 + 1,
                    jnp.maximum(_sload(maxloc, i0 + 1), _sload(ba1, j)))
            return 0

        lax.fori_loop(0, cntb, mx, 0)
        pltpu.sync_copy(maxloc, sh_max.at[pl.ds(t * SCAP * 2, SCAP * 2)])

    plsc.subcore_barrier()

    @pl.when(on0)
    def _p4():
        hdr = pubi[pl.ds(0, L)]
        nslots = hdr[0]
        cntb = hdr[2]
        nb = (cntb + L - 1) // L
        nbm = (nslots * 2 + L - 1) // L
        pltpu.sync_copy(sh_max, all16)

        def cmax(i, _):
            acc = jnp.full((L,), NEG, jnp.float32)
            for t2 in range(NT):
                acc = jnp.maximum(acc, all16[pl.ds(t2 * SCAP * 2 + i * L, L)])
            amaxg[pl.ds(i * L, L)] = acc
            return 0

        lax.fori_loop(0, nbm, cmax, 0)

        def expp(r, _):
            slv = jnp.minimum(bslot[pl.ds(r * L, L)] - 1, SCAP - 1)
            i0 = slv * 2
            am0 = plsc.load_gather(amaxg, [i0])
            am1 = plsc.load_gather(amaxg, [i0 + 1])
            ba0[pl.ds(r * L, L)] = jnp.exp(ba0[pl.ds(r * L, L)] - am0)
            ba1[pl.ds(r * L, L)] = jnp.exp(ba1[pl.ds(r * L, L)] - am1)
            return 0

        lax.fori_loop(0, nb, expp, 0)
        for i in range(SCAP * 2 // L):
            dloc[pl.ds(i * L, L)] = zf
            sxloc[pl.ds(i * L, L)] = zf

        def accj(j, _):
            i0 = jnp.minimum(_sload(bslot, j) - 1, SCAP - 1) * 2
            e0 = _sload(ba0, j)
            e1 = _sload(ba1, j)
            xsj = _sload(bxs, j)
            _sstore(dloc, i0, _sload(dloc, i0) + e0)
            _sstore(dloc, i0 + 1, _sload(dloc, i0 + 1) + e1)
            _sstore(sxloc, i0, _sload(sxloc, i0) + e0 * xsj)
            _sstore(sxloc, i0 + 1, _sload(sxloc, i0 + 1) + e1 * xsj)
            return 0

        lax.fori_loop(0, cntb, accj, 0)
        pltpu.sync_copy(dloc, sh_d.at[pl.ds(t * SCAP * 2, SCAP * 2)])
        pltpu.sync_copy(sxloc, sh_sx.at[pl.ds(t * SCAP * 2, SCAP * 2)])

    plsc.subcore_barrier()

    @pl.when(on0)
    def _p5():
        nslots = pubi[pl.ds(0, L)][0]
        nbm = (nslots * 2 + L - 1) // L
        pltpu.sync_copy(sh_d, all16)

        def csum_d(i, _):
            acc = zf
            for t2 in range(NT):
                acc = acc + all16[pl.ds(t2 * SCAP * 2 + i * L, L)]
            dg[pl.ds(i * L, L)] = acc
            return 0

        lax.fori_loop(0, nbm, csum_d, 0)
        pltpu.sync_copy(sh_sx, all16)

        def csum_s(i, _):
            acc = zf
            for t2 in range(NT):
                acc = acc + all16[pl.ds(t2 * SCAP * 2 + i * L, L)]
            sxg[pl.ds(i * L, L)] = acc
            return 0

        lax.fori_loop(0, nbm, csum_s, 0)

        n_my = (nslots + NT - 1 - t) // NT

        def slotk(k, _):
            s = t + k * NT
            i0 = s * 2
            d0 = _sload(dg, i0)
            d1 = _sload(dg, i0 + 1)
            sx0 = _sdiv(_sload(sxg, i0), d0 + 1e-16)
            sx1 = _sdiv(_sload(sxg, i0 + 1), d1 + 1e-16)
            sa0 = _sdiv(d0, d0 + 1e-16)
            sa1 = _sdiv(d1, d1 + 1e-16)
            for k2 in range(8):
                sx = sx0 if k2 < 4 else sx1
                sa = sa0 if k2 < 4 else sa1
                hv = sx * wsm[pl.ds(W1L * HC + k2 * L, L)] \
                    + sa * wsm[pl.ds(B1L * HC + k2 * L, L)] \
                    + wsm[pl.ds(BIAS1 * HC + k2 * L, L)]
                h1s[pl.ds(k * HC + k2 * L, L)] = jnp.maximum(hv, 0.0)

            def mv(c, accs):
                hval = _sload(h1s, k * HC + c)
                return tuple(accs[k2] + hval * w2lv[pl.ds(c * HC + k2 * L, L)]
                             for k2 in range(8))

            accs = tuple(wsm[pl.ds(B2L * HC + k2 * L, L)] for k2 in range(8))
            accs = lax.fori_loop(0, HC, mv, accs)
            for k2 in range(8):
                xl2acc[pl.ds(k * HC + k2 * L, L)] = accs[k2]
            pltpu.sync_copy(xl2acc.at[pl.ds(k * HC, HC)],
                            sh_xl2.at[pl.ds(s * HC, HC)])
            return 0

        lax.fori_loop(0, n_my, slotk, 0)

        @pl.when(t == 0)
        def _xr():
            accs = tuple(wsm[pl.ds(B2R * HC + k2 * L, L)] for k2 in range(8))
            for q in range(8):
                pltpu.sync_copy(w2r_h.at[pl.ds(q * L * HC, L * HC)], wbuf)

                def mv2(rr, a2):
                    hval = _sload(h1s, q * L + rr)
                    return tuple(a2[k2] + hval * wbuf[pl.ds(rr * HC + k2 * L, L)]
                                 for k2 in range(8))

                accs = lax.fori_loop(0, L, mv2, accs)
            for k2 in range(8):
                xr2acc[pl.ds(k2 * L, L)] = accs[k2]

    plsc.subcore_barrier()

    @pl.when(jnp.logical_and(on0, t == 0))
    def _p6():
        ne2 = pubi[pl.ds(0, L)][1]

        def a2j(j, carry):
            m20, m21 = carry
            s = jnp.minimum(_sload(e2slot, j), SCAP - 1)
            eav = _sload(e2ea, j)
            pltpu.sync_copy(sh_xl2.at[pl.ds(s * HC, HC)], rowbuf)
            acc0 = zf
            acc1 = zf
            for k2 in range(8):
                mv = rowbuf[pl.ds(k2 * L, L)] + xr2acc[pl.ds(k2 * L, L)] \
                    + eav * wsm[pl.ds(WE2 * HC + k2 * L, L)]
                mv = jnp.where(mv >= 0, mv, 0.2 * mv)
                prod = mv * wsm[pl.ds(ATT2 * HC + k2 * L, L)]
                if k2 < 4:
                    acc0 = acc0 + prod
                else:
                    acc1 = acc1 + prod
            a0 = jnp.sum(acc0)
            a1 = jnp.sum(acc1)
            _sstore(ba0, j, a0)
            _sstore(ba1, j, a1)
            return jnp.maximum(m20, a0), jnp.maximum(m21, a1)

        m20, m21 = lax.fori_loop(0, ne2, a2j,
                                 (jnp.float32(NEG), jnp.float32(NEG)))

        def e2j(j, carry):
            d20, d21 = carry
            e0 = jnp.exp(jnp.full((L,), _sload(ba0, j) - m20))[0]
            e1 = jnp.exp(jnp.full((L,), _sload(ba1, j) - m21))[0]
            _sstore(ba0, j, e0)
            _sstore(ba1, j, e1)
            return d20 + e0, d21 + e1

        d20, d21 = lax.fori_loop(0, ne2, e2j,
                                 (jnp.float32(0), jnp.float32(0)))
        inv0 = _sdiv(jnp.float32(1.0), d20 + 1e-16)
        inv1 = _sdiv(jnp.float32(1.0), d21 + 1e-16)

        def wsumj(j, accs):
            s = jnp.minimum(_sload(e2slot, j), SCAP - 1)
            pltpu.sync_copy(sh_xl2.at[pl.ds(s * HC, HC)], rowbuf)
            w0 = _sload(ba0, j) * inv0
            w1 = _sload(ba1, j) * inv1
            return tuple(accs[k2] + (w0 if k2 < 4 else w1)
                         * rowbuf[pl.ds(k2 * L, L)] for k2 in range(8))

        accs = tuple(wsm[pl.ds(BIAS2 * HC + k2 * L, L)] for k2 in range(8))
        accs = lax.fori_loop(0, ne2, wsumj, accs)
        for k2 in range(8):
            tvec[pl.ds(k2 * L, L)] = accs[k2]

        raccs = tuple(wsm[pl.ds(BFC * HC + k2 * L, L)] for k2 in range(8))
        for q in range(8):
            pltpu.sync_copy(wfc_h.at[pl.ds(q * L * HC, L * HC)], wbuf)

            def mv3(rr, a2):
                hval = _sload(tvec, q * L + rr)
                return tuple(a2[k2] + hval * wbuf[pl.ds(rr * HC + k2 * L, L)]
                             for k2 in range(8))

            raccs = lax.fori_loop(0, L, mv3, raccs)
        for k2 in range(8):
            rowbuf[pl.ds(k2 * L, L)] = raccs[k2]
        pltpu.sync_copy(rowbuf, out_h)


@jax.jit
def _run(src, dst, ea, x, tgtv, wsmall, w2l, w2r, wfc):
    f32 = jnp.float32
    i32 = jnp.int32
    kfn = pl.kernel(
        _body,
        out_type=jax.ShapeDtypeStruct((HC,), f32),
        compiler_params=pltpu.CompilerParams(
            needs_layout_passes=False, use_tc_tiling_on_sc=False),
        mesh=plsc.VectorSubcoreMesh(core_axis_name="c", subcore_axis_name="s"),
        scratch_types=[
            pltpu.VMEM((N,), f32),
            pltpu.VMEM((N,), i32),
            pltpu.VMEM((CHUNK,), i32),
            pltpu.VMEM((CHUNK,), i32),
            pltpu.VMEM((CHUNK,), f32),
            pltpu.VMEM((CAP_A,), i32),
            pltpu.VMEM((CAP_A,), f32),
            pltpu.VMEM((NT * CAP_A,), i32),
            pltpu.VMEM((NT * CAP_A,), f32),
            pltpu.VMEM((NT * L,), i32),
            pltpu.VMEM((NT * L,), f32),
            pltpu.VMEM((SCAP,), i32),
            pltpu.VMEM((SCAP,), i32),
            pltpu.VMEM((SCAP,), f32),
            pltpu.VMEM((CAP_B,), i32),
            pltpu.VMEM((CAP_B,), i32),
            pltpu.VMEM((CAP_B,), f32),
            pltpu.VMEM((CAP_B,), f32),
            pltpu.VMEM((CAP_B,), f32),
            pltpu.VMEM((CAP_B,), f32),
            pltpu.VMEM((CAP_B,), f32),
            pltpu.VMEM((SCAP * 2,), f32),
            pltpu.VMEM((SCAP * 2,), f32),
            pltpu.VMEM((SCAP * 2,), f32),
            pltpu.VMEM((NT * SCAP * 2,), f32),
            pltpu.VMEM((SCAP * 2,), f32),
            pltpu.VMEM((SCAP * 2,), f32),
            pltpu.VMEM((SCAP * 2,), f32),
            pltpu.VMEM((HC * HC,), f32),
            pltpu.VMEM((17 * HC,), f32),
            pltpu.VMEM((17 * HC,), f32),
            pltpu.VMEM((HC,), f32),
            pltpu.VMEM((L * HC,), f32),
            pltpu.VMEM((HC,), f32),
            pltpu.VMEM((HC,), f32),
            pltpu.VMEM((L,), i32),
            pltpu.VMEM((L,), f32),
            pltpu.VMEM((13 * HC,), f32),
            pltpu.SMEM((5 * HC,), f32),
            pltpu.VMEM_SHARED((NT * CAP_A,), i32),
            pltpu.VMEM_SHARED((NT * CAP_A,), f32),
            pltpu.VMEM_SHARED((NT * L,), i32),
            pltpu.VMEM_SHARED((NT * L,), f32),
            pltpu.VMEM_SHARED((NT * SCAP * 2,), f32),
            pltpu.VMEM_SHARED((NT * SCAP * 2,), f32),
            pltpu.VMEM_SHARED((NT * SCAP * 2,), f32),
            pltpu.VMEM_SHARED((SCAP * HC,), f32),
        ],
    )
    return kfn(src, dst, ea, x, tgtv, wsmall, w2l, w2r, wfc)


def kernel(x, edge_index, edge_attr, target_node_idx, W1l, b1l, W1r, b1r,
           We1, att1, bias1, W2l, b2l, W2r, b2r, We2, att2, bias2, Wfc, bfc):
    src = edge_index[0]
    dst = edge_index[1]
    xf = x.reshape(N).astype(jnp.float32)
    eaf = edge_attr.reshape(E).astype(jnp.float32)
    tgtv = jnp.full((L,), jnp.asarray(target_node_idx, jnp.int32))
    wsmall = jnp.concatenate([
        W1l.reshape(HC), W1r.reshape(HC), We1.reshape(HC), att1.reshape(HC),
        (b1l + b1r).reshape(HC), b1l.reshape(HC), bias1.reshape(HC),
        We2.reshape(HC), att2.reshape(HC), b2l.reshape(HC), b2r.reshape(HC),
        bias2.reshape(HC), bfc.reshape(HC)]).astype(jnp.float32)
    return _run(src.astype(jnp.int32), dst.astype(jnp.int32), eaf, xf, tgtv,
                wsmall, W2l.reshape(HC * HC).astype(jnp.float32),
                W2r.reshape(HC * HC).astype(jnp.float32),
                Wfc.reshape(HC * HC).astype(jnp.float32))
